# Initial kernel scaffold; baseline (speedup 1.0000x reference)
import jax
import jax.numpy as jnp
from jax.experimental import pallas as pl


def kernel(node, adj, edge_attr, batch, embedding, ean, etans, W1, b1, W2, b2):
    raise NotImplementedError("write your pallas kernel here")


# speedup vs baseline: 2.0472x; 2.0472x over previous
import functools

import jax
import jax.numpy as jnp
from jax import lax
from jax.experimental import pallas as pl
from jax.experimental.pallas import tpu as pltpu
from jax.experimental.pallas import tpu_sc as plsc

NODE_NUM = 100000
N = 50000
E = 800000
NUM_GRAPHS = 64
GLOVE = 100
DIM = 100
NUM_CLASS = 52

NC, NS, LANES = 2, 16, 16
NW = NC * NS
N_PAD = 50176
E_PAD = 802816
NPW = N_PAD // NW
EPW = E_PAD // NW
EPT = E_PAD // NS
BLK = 128
F = 128
CH = 32
H = 112
ACC2_ROWS = 72

_mesh = plsc.VectorSubcoreMesh(
    core_axis_name="c", subcore_axis_name="s", num_cores=NC, num_subcores=NS)

_f32 = jnp.float32
_i32 = jnp.int32


def _k0_body(emb_ref, et_ref, o0, o1, o2, o3):
    x = emb_ref[...] * et_ref[...]
    o0[...] = x[:, 0:32]
    o1[...] = x[:, 32:64]
    o2[...] = x[:, 64:96]
    o3[...] = jnp.concatenate(
        [x[:, 96:100], jnp.zeros((x.shape[0], 28), _f32)], axis=1)


def _run_k0(embedding, etans):
    blk = 1000
    grid = NODE_NUM // blk
    return pl.pallas_call(
        _k0_body,
        grid=(grid,),
        in_specs=[
            pl.BlockSpec((blk, GLOVE), lambda i: (i, 0)),
            pl.BlockSpec((blk, 1), lambda i: (i, 0)),
        ],
        out_specs=[pl.BlockSpec((blk, CH), lambda i: (i, 0))] * 4,
        out_shape=[jax.ShapeDtypeStruct((NODE_NUM, CH), _f32)] * 4,
    )(embedding, etans.reshape(NODE_NUM, 1))


def _k1_body(emb0, emb1, emb2c, emb3, node_p, dst_p, ea_p, ean, batch_p,
             h0, h1, h2, h3, w_out, g_out,
             nidx, cbuf, didx, eidx, wbuf, gbuf, sem):
    wid = lax.axis_index("s") * NC + lax.axis_index("c")
    hrefs = (h0, h1, h2, h3)
    erefs = (emb0, emb1, emb2c, emb3)
    nbase = wid * NPW

    def node_block(base, nb):
        pltpu.sync_copy(node_p.at[pl.ds(base, nb)], nidx.at[pl.ds(0, nb)])
        for c in range(4):
            pltpu.async_copy(
                erefs[c].at[nidx.at[pl.ds(0, nb)]],
                cbuf.at[pl.ds(0, nb)], sem).wait()
            pltpu.sync_copy(cbuf.at[pl.ds(0, nb)],
                            hrefs[c].at[pl.ds(base, nb)])

    def blockA(i, carry):
        node_block(nbase + i * BLK, BLK)
        return carry

    lax.fori_loop(0, NPW // BLK, blockA, 0)
    node_block(nbase + (NPW // BLK) * BLK, NPW % BLK)

    ebase = wid * EPW

    def blockB(i, carry):
        b = ebase + i * BLK
        pltpu.sync_copy(dst_p.at[pl.ds(b, BLK)], didx)
        pltpu.sync_copy(ea_p.at[pl.ds(b, BLK)], eidx)
        pltpu.async_copy(ean.at[eidx], wbuf, sem).wait()
        pltpu.async_copy(batch_p.at[didx], gbuf, sem).wait()
        pltpu.sync_copy(wbuf, w_out.at[pl.ds(b, BLK)])
        pltpu.sync_copy(gbuf, g_out.at[pl.ds(b, BLK)])
        return carry

    lax.fori_loop(0, EPW // BLK, blockB, 0)


def _run_k1(embs, node_p, dst_p, ea_p, ean, batch_p):
    out_type = ([jax.ShapeDtypeStruct((N_PAD, CH), _f32)] * 4
                + [jax.ShapeDtypeStruct((E_PAD,), _f32),
                   jax.ShapeDtypeStruct((E_PAD,), _i32)])
    k = pl.kernel(
        _k1_body,
        out_type=out_type,
        mesh=_mesh,
        compiler_params=pltpu.CompilerParams(use_tc_tiling_on_sc=False, needs_layout_passes=False),
        scratch_types=[
            pltpu.VMEM((BLK,), _i32),
            pltpu.VMEM((BLK, CH), _f32),
            pltpu.VMEM((BLK,), _i32),
            pltpu.VMEM((BLK,), _i32),
            pltpu.VMEM((BLK,), _f32),
            pltpu.VMEM((BLK,), _i32),
            pltpu.SemaphoreType.DMA,
        ],
    )
    return k(embs[0], embs[1], embs[2], embs[3], node_p, dst_p, ea_p, ean,
             batch_p)


def _k2_body(h0, h1, h2, h3, src_p, dst_p, w_arr,
             agg0, agg1, agg2, agg3,
             sidx, didx, wv, buf, zbuf, acc, sem):
    core = lax.axis_index("c")
    s = lax.axis_index("s")
    hrefs = (h0, h1, h2, h3)
    arefs = (agg0, agg1, agg2, agg3)

    def zr(k, carry):
        zbuf[k, pl.ds(0, 16)] = jnp.zeros((16,), _f32)
        zbuf[k, pl.ds(16, 16)] = jnp.zeros((16,), _f32)
        return carry

    lax.fori_loop(0, 64, zr, 0)
    zoff = s * (N_PAD // NS)

    for ci in range(2):
        def zcp(k, carry):
            pltpu.sync_copy(zbuf, acc.at[pl.ds(zoff + k * 64, 64)])
            return carry

        lax.fori_loop(0, (N_PAD // NS) // 64, zcp, 0)
        plsc.subcore_barrier()

        for c_id in range(NC):
            chunk = c_id * 2 + ci

            @pl.when(core == c_id)
            def _scatter(chunk=chunk):
                def eb(i, carry):
                    b = s * EPT + i * BLK
                    pltpu.sync_copy(src_p.at[pl.ds(b, BLK)], sidx)
                    pltpu.sync_copy(dst_p.at[pl.ds(b, BLK)], didx)
                    pltpu.sync_copy(w_arr.at[pl.ds(b, BLK)], wv)
                    pltpu.async_copy(hrefs[chunk].at[sidx], buf, sem).wait()

                    def mul(e, c2):
                        ws = plsc.load_gather(
                            wv, [jnp.full((16,), e, _i32)])
                        buf[e, pl.ds(0, 16)] = buf[e, pl.ds(0, 16)] * ws
                        buf[e, pl.ds(16, 16)] = buf[e, pl.ds(16, 16)] * ws
                        return c2

                    lax.fori_loop(0, BLK, mul, 0)
                    pltpu.sync_copy(buf, acc.at[didx], add=True)
                    return carry

                lax.fori_loop(0, EPT // BLK, eb, 0)

        plsc.subcore_barrier()

        for c_id in range(NC):
            chunk = c_id * 2 + ci

            @pl.when(core == c_id)
            def _writeback(chunk=chunk):
                def wb(k, carry):
                    off = zoff + k * 64
                    pltpu.sync_copy(acc.at[pl.ds(off, 64)],
                                    arefs[chunk].at[pl.ds(off, 64)])
                    return carry

                lax.fori_loop(0, (N_PAD // NS) // 64, wb, 0)

        plsc.subcore_barrier()


def _run_k2(hs, src_p, dst_p, w_arr):
    k = pl.kernel(
        _k2_body,
        out_type=[jax.ShapeDtypeStruct((N_PAD, CH), _f32)] * 4,
        mesh=_mesh,
        compiler_params=pltpu.CompilerParams(use_tc_tiling_on_sc=False, needs_layout_passes=False),
        scratch_types=[
            pltpu.VMEM((BLK,), _i32),
            pltpu.VMEM((BLK,), _i32),
            pltpu.VMEM((BLK,), _f32),
            pltpu.VMEM((BLK, CH), _f32),
            pltpu.VMEM((64, CH), _f32),
            pltpu.VMEM_SHARED((N_PAD, CH), _f32),
            pltpu.SemaphoreType.DMA,
        ],
    )
    return k(hs[0], hs[1], hs[2], hs[3], src_p, dst_p, w_arr)


def _k3_body(a0, a1, a2, a3, w1_ref, b1_ref, out_ref):
    x = jnp.concatenate([a0[...], a1[...], a2[...], a3[...]], axis=1)
    y = jnp.dot(x, w1_ref[...], preferred_element_type=_f32) + b1_ref[...]
    out_ref[...] = jnp.maximum(y, 0.0)


def _run_k3(aggs, W1p, b1p):
    blk = 512
    grid = N_PAD // blk
    return pl.pallas_call(
        _k3_body,
        grid=(grid,),
        in_specs=[pl.BlockSpec((blk, CH), lambda i: (i, 0))] * 4
        + [pl.BlockSpec((F, H), lambda i: (0, 0)),
           pl.BlockSpec((1, H), lambda i: (0, 0))],
        out_specs=pl.BlockSpec((blk, H), lambda i: (i, 0)),
        out_shape=jax.ShapeDtypeStruct((N_PAD, H), _f32),
    )(aggs[0], aggs[1], aggs[2], aggs[3], W1p, b1p)


def _k4_body(h1_hbm, src_p, w_arr, g_arr, batch_p,
             out2, outc,
             sidx, gv, wv, buf, zbuf2, obuf, gvn, gvn_t, acc2, accc, sem):
    core = lax.axis_index("c")
    s = lax.axis_index("s")
    wid = s * NC + core

    def zr(k, carry):
        obuf[k, pl.ds(0, 16)] = jnp.full((16,), 1.0, _f32)
        return carry

    lax.fori_loop(0, BLK, zr, 0)

    def zr2(k, carry):
        for j in range(H // 16):
            zbuf2[k, pl.ds(16 * j, 16)] = jnp.zeros((16,), _f32)
        return carry

    lax.fori_loop(0, 8, zr2, 0)

    @pl.when(s == 0)
    def _zero_acc():
        def zcp(k, carry):
            pltpu.sync_copy(zbuf2, acc2.at[pl.ds(k * 8, 8)])
            return carry

        lax.fori_loop(0, ACC2_ROWS // 8, zcp, 0)

        def zcc(k, carry):
            pltpu.sync_copy(zbuf2.at[pl.ds(0, 8), pl.ds(0, 16)],
                            accc.at[pl.ds(k * 8, 8)])
            return carry

        lax.fori_loop(0, ACC2_ROWS // 8, zcc, 0)

    plsc.subcore_barrier()

    ebase = wid * EPW

    def eb(i, carry):
        b = ebase + i * BLK
        pltpu.sync_copy(src_p.at[pl.ds(b, BLK)], sidx)
        pltpu.sync_copy(g_arr.at[pl.ds(b, BLK)], gv)
        pltpu.sync_copy(w_arr.at[pl.ds(b, BLK)], wv)
        pltpu.async_copy(h1_hbm.at[sidx], buf, sem).wait()

        def mul(e, c2):
            ws = plsc.load_gather(wv, [jnp.full((16,), e, _i32)])
            for j in range(H // 16):
                buf[e, pl.ds(16 * j, 16)] = buf[e, pl.ds(16 * j, 16)] * ws
            return c2

        lax.fori_loop(0, BLK, mul, 0)
        pltpu.sync_copy(buf, acc2.at[gv], add=True)
        return carry

    lax.fori_loop(0, EPW // BLK, eb, 0)

    nbase = wid * NPW

    def cb(i, carry):
        pltpu.sync_copy(batch_p.at[pl.ds(nbase + i * BLK, BLK)], gvn)
        pltpu.sync_copy(obuf, accc.at[gvn], add=True)
        return carry

    lax.fori_loop(0, NPW // BLK, cb, 0)
    tb = nbase + (NPW // BLK) * BLK
    pltpu.sync_copy(batch_p.at[pl.ds(tb, NPW % BLK)], gvn_t)
    pltpu.sync_copy(obuf.at[pl.ds(0, NPW % BLK)], accc.at[gvn_t], add=True)

    plsc.subcore_barrier()

    @pl.when(s == 0)
    def _writeback():
        pltpu.sync_copy(acc2.at[pl.ds(0, NUM_GRAPHS)], out2.at[core])
        pltpu.sync_copy(accc.at[pl.ds(0, NUM_GRAPHS)], outc.at[core])


def _run_k4(h1, src_p, w_arr, g_arr, batch_p):
    k = pl.kernel(
        _k4_body,
        out_type=[jax.ShapeDtypeStruct((NC, NUM_GRAPHS, H), _f32),
                  jax.ShapeDtypeStruct((NC, NUM_GRAPHS, 16), _f32)],
        mesh=_mesh,
        compiler_params=pltpu.CompilerParams(use_tc_tiling_on_sc=False, needs_layout_passes=False),
        scratch_types=[
            pltpu.VMEM((BLK,), _i32),
            pltpu.VMEM((BLK,), _i32),
            pltpu.VMEM((BLK,), _f32),
            pltpu.VMEM((BLK, H), _f32),
            pltpu.VMEM((8, H), _f32),
            pltpu.VMEM((BLK, 16), _f32),
            pltpu.VMEM((BLK,), _i32),
            pltpu.VMEM((NPW % BLK,), _i32),
            pltpu.VMEM_SHARED((ACC2_ROWS, H), _f32),
            pltpu.VMEM_SHARED((ACC2_ROWS, 16), _f32),
            pltpu.SemaphoreType.DMA,
        ],
    )
    return k(h1, src_p, w_arr, g_arr, batch_p)


def _k5_body(o2_ref, oc_ref, w2_ref, b2_ref, out_ref):
    summed = o2_ref[0] + o2_ref[1]
    cnt = oc_ref[0] + oc_ref[1]
    cnt1 = jnp.maximum(cnt[:, 0:1], 1.0)
    pooled = summed * (1.0 / cnt1)
    logits = (jnp.dot(pooled[:, :DIM], w2_ref[...],
                      preferred_element_type=_f32) + b2_ref[...])
    out_ref[...] = logits


def _run_k5(out2, outc, W2, b2):
    return pl.pallas_call(
        _k5_body,
        out_shape=jax.ShapeDtypeStruct((NUM_GRAPHS, NUM_CLASS), _f32),
    )(out2, outc, W2, b2.reshape(1, NUM_CLASS))


def kernel(node, adj, edge_attr, batch, embedding, ean, etans, W1, b1, W2,
           b2):
    node = node.astype(_i32)
    adj = adj.astype(_i32)
    edge_attr = edge_attr.astype(_i32)
    batch = batch.astype(_i32)

    epad = E_PAD - E
    npad = N_PAD - N
    src_p = jnp.concatenate([adj[0], jnp.zeros((epad,), _i32)])
    dst_p = jnp.concatenate([adj[1], jnp.full((epad,), N, _i32)])
    ea_p = jnp.concatenate([edge_attr, jnp.zeros((epad,), _i32)])
    node_p = jnp.concatenate([node, jnp.zeros((npad,), _i32)])
    batch_p = jnp.concatenate([batch, jnp.full((npad,), NUM_GRAPHS, _i32)])
    W1p = jnp.pad(W1, ((0, F - GLOVE), (0, H - DIM)))
    b1p = jnp.pad(b1, (0, H - DIM)).reshape(1, H)

    embs = _run_k0(embedding, etans)
    h_parts = _run_k1(embs, node_p, dst_p, ea_p, ean, batch_p)
    hs, w_arr, g_arr = h_parts[:4], h_parts[4], h_parts[5]
    aggs = _run_k2(hs, src_p, dst_p, w_arr)
    h1 = _run_k3(aggs, W1p, b1p)
    out2, outc = _run_k4(h1, src_p, w_arr, g_arr, batch_p)
    return _run_k5(out2, outc, W2, b2)


# speedup vs baseline: 4.1071x; 2.0062x over previous
import functools

import jax
import jax.numpy as jnp
from jax import lax
from jax.experimental import pallas as pl
from jax.experimental.pallas import tpu as pltpu
from jax.experimental.pallas import tpu_sc as plsc

NODE_NUM = 100000
N = 50000
E = 800000
NUM_GRAPHS = 64
GLOVE = 100
DIM = 100
NUM_CLASS = 52

NC, NS, LANES = 2, 16, 16
NW = NC * NS
N_PAD = 50176
E_PAD = 802816
NPW = N_PAD // NW
EPW = E_PAD // NW
EPT = E_PAD // NS
BLK = 128
F = 128
CH = 32
H = 112
ACC2_ROWS = 72

_mesh = plsc.VectorSubcoreMesh(
    core_axis_name="c", subcore_axis_name="s", num_cores=NC, num_subcores=NS)

_f32 = jnp.float32
_i32 = jnp.int32


def _k0_body(emb_ref, et_ref, o0, o1, o2, o3):
    x = emb_ref[...] * et_ref[...]
    o0[...] = x[:, 0:32]
    o1[...] = x[:, 32:64]
    o2[...] = x[:, 64:96]
    o3[...] = jnp.concatenate(
        [x[:, 96:100], jnp.zeros((x.shape[0], 28), _f32)], axis=1)


def _run_k0(embedding, etans):
    blk = 1000
    grid = NODE_NUM // blk
    return pl.pallas_call(
        _k0_body,
        grid=(grid,),
        in_specs=[
            pl.BlockSpec((blk, GLOVE), lambda i: (i, 0)),
            pl.BlockSpec((blk, 1), lambda i: (i, 0)),
        ],
        out_specs=[pl.BlockSpec((blk, CH), lambda i: (i, 0))] * 4,
        out_shape=[jax.ShapeDtypeStruct((NODE_NUM, CH), _f32)] * 4,
    )(embedding, etans.reshape(NODE_NUM, 1))


def _k1_body(emb0, emb1, emb2c, emb3, node_p, dst2, ea2, ean, batch_p,
             h0, h1, h2, h3, w_out, g_out,
             nidx, cb0, cb1, cb2, cb3, didx4, eidx4, wbuf4, gbuf4, sem):
    wid = lax.axis_index("s") * NC + lax.axis_index("c")
    hrefs = (h0, h1, h2, h3)
    erefs = (emb0, emb1, emb2c, emb3)
    cbufs = (cb0, cb1, cb2, cb3)
    nbase = wid * NPW

    def node_block(base, nb):
        pltpu.sync_copy(node_p.at[pl.ds(base, nb)], nidx.at[pl.ds(0, nb)])
        descs = [
            pltpu.async_copy(erefs[c].at[nidx.at[pl.ds(0, nb)]],
                             cbufs[c].at[pl.ds(0, nb)], sem)
            for c in range(4)]
        for c in range(4):
            descs[c].wait()
            pltpu.sync_copy(cbufs[c].at[pl.ds(0, nb)],
                            hrefs[c].at[pl.ds(base, nb)])

    def blockA(i, carry):
        node_block(nbase + i * BLK, BLK)
        return carry

    lax.fori_loop(0, NPW // BLK, blockA, 0)
    node_block(nbase + (NPW // BLK) * BLK, NPW % BLK)

    rbase = wid * (EPW // BLK)

    def blockB(i, carry):
        row = rbase + i * 4
        pltpu.sync_copy(dst2.at[pl.ds(row, 4)], didx4)
        pltpu.sync_copy(ea2.at[pl.ds(row, 4)], eidx4)
        descs = []
        for k in range(4):
            descs.append(pltpu.async_copy(
                ean.at[eidx4.at[k]], wbuf4.at[k], sem))
            descs.append(pltpu.async_copy(
                batch_p.at[didx4.at[k]], gbuf4.at[k], sem))
        for d in descs:
            d.wait()
        pltpu.sync_copy(wbuf4, w_out.at[pl.ds(row, 4)])
        pltpu.sync_copy(gbuf4, g_out.at[pl.ds(row, 4)])
        return carry

    lax.fori_loop(0, EPW // BLK // 4, blockB, 0)


def _run_k1(embs, node_p, dst2, ea2, ean, batch_p):
    out_type = ([jax.ShapeDtypeStruct((N_PAD, CH), _f32)] * 4
                + [jax.ShapeDtypeStruct((E_PAD // BLK, BLK), _f32),
                   jax.ShapeDtypeStruct((E_PAD // BLK, BLK), _i32)])
    k = pl.kernel(
        _k1_body,
        out_type=out_type,
        mesh=_mesh,
        compiler_params=pltpu.CompilerParams(use_tc_tiling_on_sc=False, needs_layout_passes=False),
        scratch_types=[
            pltpu.VMEM((BLK,), _i32),
            pltpu.VMEM((BLK, CH), _f32),
            pltpu.VMEM((BLK, CH), _f32),
            pltpu.VMEM((BLK, CH), _f32),
            pltpu.VMEM((BLK, CH), _f32),
            pltpu.VMEM((4, BLK), _i32),
            pltpu.VMEM((4, BLK), _i32),
            pltpu.VMEM((4, BLK), _f32),
            pltpu.VMEM((4, BLK), _i32),
            pltpu.SemaphoreType.DMA,
        ],
    )
    return k(embs[0], embs[1], embs[2], embs[3], node_p, dst2, ea2, ean,
             batch_p)


def _mul_rows(buf, wref, krow, g, nv):
    w16 = wref[krow, pl.ds(g * 16, 16)]
    dn = lax.GatherDimensionNumbers(
        offset_dims=(), collapsed_slice_dims=(0,), start_index_map=(0,))
    for j in range(16):
        e = g * 16 + j
        ws = lax.gather(w16, jnp.full((16, 1), j, _i32), dn, (1,),
                        mode=lax.GatherScatterMode.PROMISE_IN_BOUNDS)
        for v in range(nv):
            buf[e, pl.ds(16 * v, 16)] = buf[e, pl.ds(16 * v, 16)] * ws


def _k2_body(h0, h1, h2, h3, src2, dst2, w2,
             agg0, agg1, agg2, agg3,
             sidx4, didx4, wv4, bufa, bufb, zbuf, acc, sema, semb):
    core = lax.axis_index("c")
    s = lax.axis_index("s")
    hrefs = (h0, h1, h2, h3)
    arefs = (agg0, agg1, agg2, agg3)
    bufs = (bufa, bufb)
    sems = (sema, semb)

    def zr(k, carry):
        zbuf[k, pl.ds(0, 16)] = jnp.zeros((16,), _f32)
        zbuf[k, pl.ds(16, 16)] = jnp.zeros((16,), _f32)
        return carry

    lax.fori_loop(0, 64, zr, 0)
    zoff = s * (N_PAD // NS)
    rbase = s * (EPT // BLK)

    for ci in range(2):
        def zcp(k, carry):
            pltpu.sync_copy(zbuf, acc.at[pl.ds(zoff + k * 64, 64)])
            return carry

        lax.fori_loop(0, (N_PAD // NS) // 64, zcp, 0)
        plsc.subcore_barrier()

        for c_id in range(NC):
            chunk = c_id * 2 + ci

            @pl.when(core == c_id)
            def _scatter(chunk=chunk):
                def sb(i, carry):
                    row = rbase + i * 4
                    pltpu.sync_copy(src2.at[pl.ds(row, 4)], sidx4)
                    pltpu.sync_copy(dst2.at[pl.ds(row, 4)], didx4)
                    pltpu.sync_copy(w2.at[pl.ds(row, 4)], wv4)
                    d = pltpu.async_copy(
                        hrefs[chunk].at[sidx4.at[0]], bufs[0], sems[0])
                    for k in range(4):
                        d.wait()
                        if k < 3:
                            d = pltpu.async_copy(
                                hrefs[chunk].at[sidx4.at[k + 1]],
                                bufs[(k + 1) % 2], sems[(k + 1) % 2])

                        def mg(g, c2, k=k):
                            _mul_rows(bufs[k % 2], wv4, k, g, 2)
                            return c2

                        lax.fori_loop(0, 8, mg, 0)
                        pltpu.sync_copy(bufs[k % 2], acc.at[didx4.at[k]],
                                        add=True)
                    return carry

                lax.fori_loop(0, EPT // BLK // 4, sb, 0)

        plsc.subcore_barrier()

        for c_id in range(NC):
            chunk = c_id * 2 + ci

            @pl.when(core == c_id)
            def _writeback(chunk=chunk):
                def wb(k, carry):
                    off = zoff + k * 64
                    pltpu.sync_copy(acc.at[pl.ds(off, 64)],
                                    arefs[chunk].at[pl.ds(off, 64)])
                    return carry

                lax.fori_loop(0, (N_PAD // NS) // 64, wb, 0)

        plsc.subcore_barrier()


def _run_k2(hs, src2, dst2, w2):
    k = pl.kernel(
        _k2_body,
        out_type=[jax.ShapeDtypeStruct((N_PAD, CH), _f32)] * 4,
        mesh=_mesh,
        compiler_params=pltpu.CompilerParams(use_tc_tiling_on_sc=False, needs_layout_passes=False),
        scratch_types=[
            pltpu.VMEM((4, BLK), _i32),
            pltpu.VMEM((4, BLK), _i32),
            pltpu.VMEM((4, BLK), _f32),
            pltpu.VMEM((BLK, CH), _f32),
            pltpu.VMEM((BLK, CH), _f32),
            pltpu.VMEM((64, CH), _f32),
            pltpu.VMEM_SHARED((N_PAD, CH), _f32),
            pltpu.SemaphoreType.DMA,
            pltpu.SemaphoreType.DMA,
        ],
    )
    return k(hs[0], hs[1], hs[2], hs[3], src2, dst2, w2)


def _k3_body(a0, a1, a2, a3, w1_ref, b1_ref, out_ref):
    x = jnp.concatenate([a0[...], a1[...], a2[...], a3[...]], axis=1)
    y = jnp.dot(x, w1_ref[...], preferred_element_type=_f32) + b1_ref[...]
    out_ref[...] = jnp.maximum(y, 0.0)


def _run_k3(aggs, W1p, b1p):
    blk = 512
    grid = N_PAD // blk
    return pl.pallas_call(
        _k3_body,
        grid=(grid,),
        in_specs=[pl.BlockSpec((blk, CH), lambda i: (i, 0))] * 4
        + [pl.BlockSpec((F, H), lambda i: (0, 0)),
           pl.BlockSpec((1, H), lambda i: (0, 0))],
        out_specs=pl.BlockSpec((blk, H), lambda i: (i, 0)),
        out_shape=jax.ShapeDtypeStruct((N_PAD, H), _f32),
    )(aggs[0], aggs[1], aggs[2], aggs[3], W1p, b1p)


def _k4_body(h1_hbm, src2, w2, g2, batch_p,
             out2, outc,
             sidx4, gv4, wv4, bufa, bufb, zbuf2, obuf, gvn, gvn_t,
             acc2, accc, sema, semb):
    core = lax.axis_index("c")
    s = lax.axis_index("s")
    wid = s * NC + core
    bufs = (bufa, bufb)
    sems = (sema, semb)

    def zr(k, carry):
        obuf[k, pl.ds(0, 16)] = jnp.full((16,), 1.0, _f32)
        return carry

    lax.fori_loop(0, BLK, zr, 0)

    def zr2(k, carry):
        for j in range(H // 16):
            zbuf2[k, pl.ds(16 * j, 16)] = jnp.zeros((16,), _f32)
        return carry

    lax.fori_loop(0, 8, zr2, 0)

    @pl.when(s == 0)
    def _zero_acc():
        def zcp(k, carry):
            pltpu.sync_copy(zbuf2, acc2.at[pl.ds(k * 8, 8)])
            return carry

        lax.fori_loop(0, ACC2_ROWS // 8, zcp, 0)

        def zcc(k, carry):
            pltpu.sync_copy(zbuf2.at[pl.ds(0, 8), pl.ds(0, 16)],
                            accc.at[pl.ds(k * 8, 8)])
            return carry

        lax.fori_loop(0, ACC2_ROWS // 8, zcc, 0)

    plsc.subcore_barrier()

    rbase = wid * (EPW // BLK)

    def eb(i, carry):
        row = rbase + i * 4
        pltpu.sync_copy(src2.at[pl.ds(row, 4)], sidx4)
        pltpu.sync_copy(g2.at[pl.ds(row, 4)], gv4)
        pltpu.sync_copy(w2.at[pl.ds(row, 4)], wv4)
        d = pltpu.async_copy(h1_hbm.at[sidx4.at[0]], bufs[0], sems[0])
        for k in range(4):
            d.wait()
            if k < 3:
                d = pltpu.async_copy(h1_hbm.at[sidx4.at[k + 1]],
                                     bufs[(k + 1) % 2], sems[(k + 1) % 2])

            def mg(g, c2, k=k):
                _mul_rows(bufs[k % 2], wv4, k, g, H // 16)
                return c2

            lax.fori_loop(0, 8, mg, 0)
            pltpu.sync_copy(bufs[k % 2], acc2.at[gv4.at[k]], add=True)
        return carry

    lax.fori_loop(0, EPW // BLK // 4, eb, 0)

    nbase = wid * NPW

    def cb(i, carry):
        pltpu.sync_copy(batch_p.at[pl.ds(nbase + i * BLK, BLK)], gvn)
        pltpu.sync_copy(obuf, accc.at[gvn], add=True)
        return carry

    lax.fori_loop(0, NPW // BLK, cb, 0)
    tb = nbase + (NPW // BLK) * BLK
    pltpu.sync_copy(batch_p.at[pl.ds(tb, NPW % BLK)], gvn_t)
    pltpu.sync_copy(obuf.at[pl.ds(0, NPW % BLK)], accc.at[gvn_t], add=True)

    plsc.subcore_barrier()

    @pl.when(s == 0)
    def _writeback():
        pltpu.sync_copy(acc2.at[pl.ds(0, NUM_GRAPHS)], out2.at[core])
        pltpu.sync_copy(accc.at[pl.ds(0, NUM_GRAPHS)], outc.at[core])


def _run_k4(h1, src2, w2, g2, batch_p):
    k = pl.kernel(
        _k4_body,
        out_type=[jax.ShapeDtypeStruct((NC, NUM_GRAPHS, H), _f32),
                  jax.ShapeDtypeStruct((NC, NUM_GRAPHS, 16), _f32)],
        mesh=_mesh,
        compiler_params=pltpu.CompilerParams(use_tc_tiling_on_sc=False, needs_layout_passes=False),
        scratch_types=[
            pltpu.VMEM((4, BLK), _i32),
            pltpu.VMEM((4, BLK), _i32),
            pltpu.VMEM((4, BLK), _f32),
            pltpu.VMEM((BLK, H), _f32),
            pltpu.VMEM((BLK, H), _f32),
            pltpu.VMEM((8, H), _f32),
            pltpu.VMEM((BLK, 16), _f32),
            pltpu.VMEM((BLK,), _i32),
            pltpu.VMEM((NPW % BLK,), _i32),
            pltpu.VMEM_SHARED((ACC2_ROWS, H), _f32),
            pltpu.VMEM_SHARED((ACC2_ROWS, 16), _f32),
            pltpu.SemaphoreType.DMA,
            pltpu.SemaphoreType.DMA,
        ],
    )
    return k(h1, src2, w2, g2, batch_p)


def _k5_body(o2_ref, oc_ref, w2_ref, b2_ref, out_ref):
    summed = o2_ref[0] + o2_ref[1]
    cnt = oc_ref[0] + oc_ref[1]
    cnt1 = jnp.maximum(cnt[:, 0:1], 1.0)
    pooled = summed * (1.0 / cnt1)
    logits = (jnp.dot(pooled[:, :DIM], w2_ref[...],
                      preferred_element_type=_f32) + b2_ref[...])
    out_ref[...] = logits


def _run_k5(out2, outc, W2, b2):
    return pl.pallas_call(
        _k5_body,
        out_shape=jax.ShapeDtypeStruct((NUM_GRAPHS, NUM_CLASS), _f32),
    )(out2, outc, W2, b2.reshape(1, NUM_CLASS))


def kernel(node, adj, edge_attr, batch, embedding, ean, etans, W1, b1, W2,
           b2):
    node = node.astype(_i32)
    adj = adj.astype(_i32)
    edge_attr = edge_attr.astype(_i32)
    batch = batch.astype(_i32)

    epad = E_PAD - E
    npad = N_PAD - N
    src2 = jnp.concatenate([adj[0], jnp.zeros((epad,), _i32)]).reshape(
        E_PAD // BLK, BLK)
    dst2 = jnp.concatenate([adj[1], jnp.full((epad,), N, _i32)]).reshape(
        E_PAD // BLK, BLK)
    ea2 = jnp.concatenate([edge_attr, jnp.zeros((epad,), _i32)]).reshape(
        E_PAD // BLK, BLK)
    node_p = jnp.concatenate([node, jnp.zeros((npad,), _i32)])
    batch_p = jnp.concatenate([batch, jnp.full((npad,), NUM_GRAPHS, _i32)])
    W1p = jnp.pad(W1, ((0, F - GLOVE), (0, H - DIM)))
    b1p = jnp.pad(b1, (0, H - DIM)).reshape(1, H)

    embs = _run_k0(embedding, etans)
    h_parts = _run_k1(embs, node_p, dst2, ea2, ean, batch_p)
    hs, w2, g2 = h_parts[:4], h_parts[4], h_parts[5]
    aggs = _run_k2(hs, src2, dst2, w2)
    h1 = _run_k3(aggs, W1p, b1p)
    out2, outc = _run_k4(h1, src2, w2, g2, batch_p)
    return _run_k5(out2, outc, W2, b2)


# speedup vs baseline: 4.4259x; 1.0776x over previous
import functools

import jax
import jax.numpy as jnp
from jax import lax
from jax.experimental import pallas as pl
from jax.experimental.pallas import tpu as pltpu
from jax.experimental.pallas import tpu_sc as plsc

NODE_NUM = 100000
N = 50000
E = 800000
NUM_GRAPHS = 64
GLOVE = 100
DIM = 100
NUM_CLASS = 52

NC, NS, LANES = 2, 16, 16
NW = NC * NS
N_PAD = 50176
E_PAD = 802816
NPW = N_PAD // NW
EPW = E_PAD // NW
EPT = E_PAD // NS
BLK = 128
F = 128
CH = 32
H = 112
ACC2_ROWS = 72

_mesh = plsc.VectorSubcoreMesh(
    core_axis_name="c", subcore_axis_name="s", num_cores=NC, num_subcores=NS)

_f32 = jnp.float32
_i32 = jnp.int32


def _k0_body(emb_ref, et_ref, o0, o1, o2, o3):
    x = emb_ref[...] * et_ref[...]
    o0[...] = x[:, 0:32]
    o1[...] = x[:, 32:64]
    o2[...] = x[:, 64:96]
    o3[...] = jnp.concatenate(
        [x[:, 96:100], jnp.zeros((x.shape[0], 28), _f32)], axis=1)


def _run_k0(embedding, etans):
    blk = 1000
    grid = NODE_NUM // blk
    return pl.pallas_call(
        _k0_body,
        grid=(grid,),
        in_specs=[
            pl.BlockSpec((blk, GLOVE), lambda i: (i, 0)),
            pl.BlockSpec((blk, 1), lambda i: (i, 0)),
        ],
        out_specs=[pl.BlockSpec((blk, CH), lambda i: (i, 0))] * 4,
        out_shape=[jax.ShapeDtypeStruct((NODE_NUM, CH), _f32)] * 4,
    )(embedding, etans.reshape(NODE_NUM, 1))


def _k1_body(emb0, emb1, emb2c, emb3, node_p, dst2, ea2, ean, batch_p,
             h0, h1, h2, h3, w_out, g_out,
             nidx, cb0, cb1, cb2, cb3, didx4, eidx4, wbuf4, gbuf4, sem):
    wid = lax.axis_index("s") * NC + lax.axis_index("c")
    hrefs = (h0, h1, h2, h3)
    erefs = (emb0, emb1, emb2c, emb3)
    cbufs = (cb0, cb1, cb2, cb3)
    nbase = wid * NPW

    def node_block(base, nb):
        pltpu.sync_copy(node_p.at[pl.ds(base, nb)], nidx.at[pl.ds(0, nb)])
        descs = [
            pltpu.async_copy(erefs[c].at[nidx.at[pl.ds(0, nb)]],
                             cbufs[c].at[pl.ds(0, nb)], sem)
            for c in range(4)]
        for c in range(4):
            descs[c].wait()
            pltpu.sync_copy(cbufs[c].at[pl.ds(0, nb)],
                            hrefs[c].at[pl.ds(base, nb)])

    def blockA(i, carry):
        node_block(nbase + i * BLK, BLK)
        return carry

    lax.fori_loop(0, NPW // BLK, blockA, 0)
    node_block(nbase + (NPW // BLK) * BLK, NPW % BLK)

    rbase = wid * (EPW // BLK)

    def blockB(i, carry):
        row = rbase + i * 4
        pltpu.sync_copy(dst2.at[pl.ds(row, 4)], didx4)
        pltpu.sync_copy(ea2.at[pl.ds(row, 4)], eidx4)
        descs = []
        for k in range(4):
            descs.append(pltpu.async_copy(
                ean.at[eidx4.at[k]], wbuf4.at[k], sem))
            descs.append(pltpu.async_copy(
                batch_p.at[didx4.at[k]], gbuf4.at[k], sem))
        for d in descs:
            d.wait()
        pltpu.sync_copy(wbuf4, w_out.at[pl.ds(row, 4)])
        pltpu.sync_copy(gbuf4, g_out.at[pl.ds(row, 4)])
        return carry

    lax.fori_loop(0, EPW // BLK // 4, blockB, 0)


def _run_k1(embs, node_p, dst2, ea2, ean, batch_p):
    out_type = ([jax.ShapeDtypeStruct((N_PAD, CH), _f32)] * 4
                + [jax.ShapeDtypeStruct((E_PAD // BLK, BLK), _f32),
                   jax.ShapeDtypeStruct((E_PAD // BLK, BLK), _i32)])
    k = pl.kernel(
        _k1_body,
        out_type=out_type,
        mesh=_mesh,
        compiler_params=pltpu.CompilerParams(use_tc_tiling_on_sc=False, needs_layout_passes=False),
        scratch_types=[
            pltpu.VMEM((BLK,), _i32),
            pltpu.VMEM((BLK, CH), _f32),
            pltpu.VMEM((BLK, CH), _f32),
            pltpu.VMEM((BLK, CH), _f32),
            pltpu.VMEM((BLK, CH), _f32),
            pltpu.VMEM((4, BLK), _i32),
            pltpu.VMEM((4, BLK), _i32),
            pltpu.VMEM((4, BLK), _f32),
            pltpu.VMEM((4, BLK), _i32),
            pltpu.SemaphoreType.DMA,
        ],
    )
    return k(embs[0], embs[1], embs[2], embs[3], node_p, dst2, ea2, ean,
             batch_p)


def _mul_rows(buf, wref, krow, g, nv):
    w16 = wref[krow, pl.ds(g * 16, 16)]
    dn = lax.GatherDimensionNumbers(
        offset_dims=(), collapsed_slice_dims=(0,), start_index_map=(0,))
    for j in range(16):
        e = g * 16 + j
        ws = lax.gather(w16, jnp.full((16, 1), j, _i32), dn, (1,),
                        mode=lax.GatherScatterMode.PROMISE_IN_BOUNDS)
        for v in range(nv):
            buf[e, pl.ds(16 * v, 16)] = buf[e, pl.ds(16 * v, 16)] * ws


def _k2_body(h0, h1, h2, h3, src2, dst2, w2,
             agg0, agg1, agg2, agg3,
             sidx4, didx4, wv4, bufa, bufb, zbuf, acc,
             semga, semgb, semsa, semsb):
    core = lax.axis_index("c")
    s = lax.axis_index("s")
    hrefs = (h0, h1, h2, h3)
    arefs = (agg0, agg1, agg2, agg3)
    bufs = (bufa, bufb)
    semG = (semga, semgb)
    semS = (semsa, semsb)

    def zr(k, carry):
        zbuf[k, pl.ds(0, 16)] = jnp.zeros((16,), _f32)
        zbuf[k, pl.ds(16, 16)] = jnp.zeros((16,), _f32)
        return carry

    lax.fori_loop(0, 64, zr, 0)
    zoff = s * (N_PAD // NS)
    rbase = s * (EPT // BLK)

    for ci in range(2):
        def zcp(k, carry):
            pltpu.sync_copy(zbuf, acc.at[pl.ds(zoff + k * 64, 64)])
            return carry

        lax.fori_loop(0, (N_PAD // NS) // 64, zcp, 0)
        plsc.subcore_barrier()

        for c_id in range(NC):
            chunk = c_id * 2 + ci

            @pl.when(core == c_id)
            def _scatter(chunk=chunk):
                hdummy = hrefs[chunk].at[pl.ds(0, BLK)]

                def sb(i, carry):
                    row = rbase + i * 4
                    pltpu.sync_copy(src2.at[pl.ds(row, 4)], sidx4)
                    pltpu.sync_copy(dst2.at[pl.ds(row, 4)], didx4)
                    pltpu.sync_copy(w2.at[pl.ds(row, 4)], wv4)

                    @pl.when(i > 0)
                    def _drain0():
                        pltpu.make_async_copy(hdummy, bufs[0],
                                              semS[0]).wait()

                    pltpu.async_copy(hrefs[chunk].at[sidx4.at[0]],
                                     bufs[0], semG[0])
                    for k in range(4):
                        if k < 3:
                            nb = (k + 1) % 2
                            if k == 0:
                                @pl.when(i > 0)
                                def _drain1():
                                    pltpu.make_async_copy(
                                        hdummy, bufs[1], semS[1]).wait()
                            else:
                                pltpu.make_async_copy(
                                    hdummy, bufs[nb], semS[nb]).wait()
                            pltpu.async_copy(
                                hrefs[chunk].at[sidx4.at[k + 1]],
                                bufs[nb], semG[nb])
                        pltpu.make_async_copy(hdummy, bufs[k % 2],
                                              semG[k % 2]).wait()

                        def mg(g, c2, k=k):
                            _mul_rows(bufs[k % 2], wv4, k, g, 2)
                            return c2

                        lax.fori_loop(0, 8, mg, 0)
                        pltpu.async_copy(bufs[k % 2], acc.at[didx4.at[k]],
                                         semS[k % 2], add=True)
                    return carry

                lax.fori_loop(0, EPT // BLK // 4, sb, 0)
                pltpu.make_async_copy(hdummy, bufs[0], semS[0]).wait()
                pltpu.make_async_copy(hdummy, bufs[1], semS[1]).wait()

        plsc.subcore_barrier()

        for c_id in range(NC):
            chunk = c_id * 2 + ci

            @pl.when(core == c_id)
            def _writeback(chunk=chunk):
                def wb(k, carry):
                    off = zoff + k * 64
                    pltpu.sync_copy(acc.at[pl.ds(off, 64)],
                                    arefs[chunk].at[pl.ds(off, 64)])
                    return carry

                lax.fori_loop(0, (N_PAD // NS) // 64, wb, 0)

        plsc.subcore_barrier()


def _run_k2(hs, src2, dst2, w2):
    k = pl.kernel(
        _k2_body,
        out_type=[jax.ShapeDtypeStruct((N_PAD, CH), _f32)] * 4,
        mesh=_mesh,
        compiler_params=pltpu.CompilerParams(use_tc_tiling_on_sc=False, needs_layout_passes=False),
        scratch_types=[
            pltpu.VMEM((4, BLK), _i32),
            pltpu.VMEM((4, BLK), _i32),
            pltpu.VMEM((4, BLK), _f32),
            pltpu.VMEM((BLK, CH), _f32),
            pltpu.VMEM((BLK, CH), _f32),
            pltpu.VMEM((64, CH), _f32),
            pltpu.VMEM_SHARED((N_PAD, CH), _f32),
            pltpu.SemaphoreType.DMA,
            pltpu.SemaphoreType.DMA,
            pltpu.SemaphoreType.DMA,
            pltpu.SemaphoreType.DMA,
        ],
    )
    return k(hs[0], hs[1], hs[2], hs[3], src2, dst2, w2)


def _k3_body(a0, a1, a2, a3, w1_ref, b1_ref, out_ref):
    x = jnp.concatenate([a0[...], a1[...], a2[...], a3[...]], axis=1)
    y = jnp.dot(x, w1_ref[...], preferred_element_type=_f32) + b1_ref[...]
    out_ref[...] = jnp.maximum(y, 0.0)


def _run_k3(aggs, W1p, b1p):
    blk = 512
    grid = N_PAD // blk
    return pl.pallas_call(
        _k3_body,
        grid=(grid,),
        in_specs=[pl.BlockSpec((blk, CH), lambda i: (i, 0))] * 4
        + [pl.BlockSpec((F, H), lambda i: (0, 0)),
           pl.BlockSpec((1, H), lambda i: (0, 0))],
        out_specs=pl.BlockSpec((blk, H), lambda i: (i, 0)),
        out_shape=jax.ShapeDtypeStruct((N_PAD, H), _f32),
    )(aggs[0], aggs[1], aggs[2], aggs[3], W1p, b1p)


def _k4_body(h1_hbm, src2, w2, g2, batch_p,
             out2, outc,
             sidx4, gv4, wv4, bufa, bufb, zbuf2, obuf, gvn, gvn_t,
             acc2, accc, semga, semgb, semsa, semsb):
    core = lax.axis_index("c")
    s = lax.axis_index("s")
    wid = s * NC + core
    bufs = (bufa, bufb)
    semG = (semga, semgb)
    semS = (semsa, semsb)
    hdummy = h1_hbm.at[pl.ds(0, BLK)]

    def zr(k, carry):
        obuf[k, pl.ds(0, 16)] = jnp.full((16,), 1.0, _f32)
        return carry

    lax.fori_loop(0, BLK, zr, 0)

    def zr2(k, carry):
        for j in range(H // 16):
            zbuf2[k, pl.ds(16 * j, 16)] = jnp.zeros((16,), _f32)
        return carry

    lax.fori_loop(0, 8, zr2, 0)

    @pl.when(s == 0)
    def _zero_acc():
        def zcp(k, carry):
            pltpu.sync_copy(zbuf2, acc2.at[pl.ds(k * 8, 8)])
            return carry

        lax.fori_loop(0, ACC2_ROWS // 8, zcp, 0)

        def zcc(k, carry):
            pltpu.sync_copy(zbuf2.at[pl.ds(0, 8), pl.ds(0, 16)],
                            accc.at[pl.ds(k * 8, 8)])
            return carry

        lax.fori_loop(0, ACC2_ROWS // 8, zcc, 0)

    plsc.subcore_barrier()

    rbase = wid * (EPW // BLK)

    def eb(i, carry):
        row = rbase + i * 4
        pltpu.sync_copy(src2.at[pl.ds(row, 4)], sidx4)
        pltpu.sync_copy(g2.at[pl.ds(row, 4)], gv4)
        pltpu.sync_copy(w2.at[pl.ds(row, 4)], wv4)

        @pl.when(i > 0)
        def _drain0():
            pltpu.make_async_copy(hdummy, bufs[0], semS[0]).wait()

        pltpu.async_copy(h1_hbm.at[sidx4.at[0]], bufs[0], semG[0])
        for k in range(4):
            if k < 3:
                nb = (k + 1) % 2
                if k == 0:
                    @pl.when(i > 0)
                    def _drain1():
                        pltpu.make_async_copy(hdummy, bufs[1],
                                              semS[1]).wait()
                else:
                    pltpu.make_async_copy(hdummy, bufs[nb], semS[nb]).wait()
                pltpu.async_copy(h1_hbm.at[sidx4.at[k + 1]], bufs[nb],
                                 semG[nb])
            pltpu.make_async_copy(hdummy, bufs[k % 2], semG[k % 2]).wait()

            def mg(g, c2, k=k):
                _mul_rows(bufs[k % 2], wv4, k, g, H // 16)
                return c2

            lax.fori_loop(0, 8, mg, 0)
            pltpu.async_copy(bufs[k % 2], acc2.at[gv4.at[k]], semS[k % 2],
                             add=True)
        return carry

    lax.fori_loop(0, EPW // BLK // 4, eb, 0)
    pltpu.make_async_copy(hdummy, bufs[0], semS[0]).wait()
    pltpu.make_async_copy(hdummy, bufs[1], semS[1]).wait()

    nbase = wid * NPW

    def cb(i, carry):
        pltpu.sync_copy(batch_p.at[pl.ds(nbase + i * BLK, BLK)], gvn)
        pltpu.sync_copy(obuf, accc.at[gvn], add=True)
        return carry

    lax.fori_loop(0, NPW // BLK, cb, 0)
    tb = nbase + (NPW // BLK) * BLK
    pltpu.sync_copy(batch_p.at[pl.ds(tb, NPW % BLK)], gvn_t)
    pltpu.sync_copy(obuf.at[pl.ds(0, NPW % BLK)], accc.at[gvn_t], add=True)

    plsc.subcore_barrier()

    @pl.when(s == 0)
    def _writeback():
        pltpu.sync_copy(acc2.at[pl.ds(0, NUM_GRAPHS)], out2.at[core])
        pltpu.sync_copy(accc.at[pl.ds(0, NUM_GRAPHS)], outc.at[core])


def _run_k4(h1, src2, w2, g2, batch_p):
    k = pl.kernel(
        _k4_body,
        out_type=[jax.ShapeDtypeStruct((NC, NUM_GRAPHS, H), _f32),
                  jax.ShapeDtypeStruct((NC, NUM_GRAPHS, 16), _f32)],
        mesh=_mesh,
        compiler_params=pltpu.CompilerParams(use_tc_tiling_on_sc=False, needs_layout_passes=False),
        scratch_types=[
            pltpu.VMEM((4, BLK), _i32),
            pltpu.VMEM((4, BLK), _i32),
            pltpu.VMEM((4, BLK), _f32),
            pltpu.VMEM((BLK, H), _f32),
            pltpu.VMEM((BLK, H), _f32),
            pltpu.VMEM((8, H), _f32),
            pltpu.VMEM((BLK, 16), _f32),
            pltpu.VMEM((BLK,), _i32),
            pltpu.VMEM((NPW % BLK,), _i32),
            pltpu.VMEM_SHARED((ACC2_ROWS, H), _f32),
            pltpu.VMEM_SHARED((ACC2_ROWS, 16), _f32),
            pltpu.SemaphoreType.DMA,
            pltpu.SemaphoreType.DMA,
            pltpu.SemaphoreType.DMA,
            pltpu.SemaphoreType.DMA,
        ],
    )
    return k(h1, src2, w2, g2, batch_p)


def _k5_body(o2_ref, oc_ref, w2_ref, b2_ref, out_ref):
    summed = o2_ref[0] + o2_ref[1]
    cnt = oc_ref[0] + oc_ref[1]
    cnt1 = jnp.maximum(cnt[:, 0:1], 1.0)
    pooled = summed * (1.0 / cnt1)
    logits = (jnp.dot(pooled[:, :DIM], w2_ref[...],
                      preferred_element_type=_f32) + b2_ref[...])
    out_ref[...] = logits


def _run_k5(out2, outc, W2, b2):
    return pl.pallas_call(
        _k5_body,
        out_shape=jax.ShapeDtypeStruct((NUM_GRAPHS, NUM_CLASS), _f32),
    )(out2, outc, W2, b2.reshape(1, NUM_CLASS))


def kernel(node, adj, edge_attr, batch, embedding, ean, etans, W1, b1, W2,
           b2):
    node = node.astype(_i32)
    adj = adj.astype(_i32)
    edge_attr = edge_attr.astype(_i32)
    batch = batch.astype(_i32)

    epad = E_PAD - E
    npad = N_PAD - N
    src2 = jnp.concatenate([adj[0], jnp.zeros((epad,), _i32)]).reshape(
        E_PAD // BLK, BLK)
    dst2 = jnp.concatenate([adj[1], jnp.full((epad,), N, _i32)]).reshape(
        E_PAD // BLK, BLK)
    ea2 = jnp.concatenate([edge_attr, jnp.zeros((epad,), _i32)]).reshape(
        E_PAD // BLK, BLK)
    node_p = jnp.concatenate([node, jnp.zeros((npad,), _i32)])
    batch_p = jnp.concatenate([batch, jnp.full((npad,), NUM_GRAPHS, _i32)])
    W1p = jnp.pad(W1, ((0, F - GLOVE), (0, H - DIM)))
    b1p = jnp.pad(b1, (0, H - DIM)).reshape(1, H)

    embs = _run_k0(embedding, etans)
    h_parts = _run_k1(embs, node_p, dst2, ea2, ean, batch_p)
    hs, w2, g2 = h_parts[:4], h_parts[4], h_parts[5]
    aggs = _run_k2(hs, src2, dst2, w2)
    h1 = _run_k3(aggs, W1p, b1p)
    out2, outc = _run_k4(h1, src2, w2, g2, batch_p)
    return _run_k5(out2, outc, W2, b2)
